# baseline (speedup 1.0000x reference)
import functools

import jax
import jax.numpy as jnp
from jax import lax
from jax.experimental import pallas as pl
from jax.experimental.pallas import tpu as pltpu
from jax.experimental.pallas import tpu_sc as plsc

_NC, _NS, _L = 2, 16, 16
_NW = _NC * _NS


def _make_sc_add(B, S, D):
    CH = 16
    SPW = S // _NW
    NCH = SPW // CH
    G = NCH * B
    NR = 4
    UNROLL = 2 * B
    HALF = NR // 2
    assert S % _NW == 0 and SPW % CH == 0 and G % UNROLL == 0
    assert B == 4 and UNROLL % NR == 0 and D % _L == 0

    mesh = plsc.VectorSubcoreMesh(core_axis_name="c", subcore_axis_name="s")

    @functools.partial(
        pl.kernel,
        out_type=jax.ShapeDtypeStruct((B * S, D), jnp.float32),
        mesh=mesh,
        compiler_params=pltpu.CompilerParams(use_tc_tiling_on_sc=True),
        scratch_types=(
            [pltpu.VMEM((CH, D), jnp.float32) for _ in range(NR + 2)]
            + [pltpu.SemaphoreType.DMA for _ in range(2 * NR + 2)]
        ),
    )
    def sc_add(x_hbm, pos_hbm, out_hbm, *refs):
        bufs = refs[0:NR]
        pbufs = refs[NR:NR + 2]
        sxs = refs[NR + 2:2 * NR + 2]
        sos = refs[2 * NR + 2:3 * NR + 2]
        sps = refs[3 * NR + 2:3 * NR + 4]

        w = lax.axis_index("s") * _NC + lax.axis_index("c")
        s_row = w * SPW

        def row_off(g):
            return lax.rem(g, B) * S + s_row + lax.div(g, B) * CH

        def start_x(g, buf, sem):
            pltpu.make_async_copy(
                x_hbm.at[pl.ds(row_off(g), CH)], buf, sem).start()

        def start_pos(c, buf, sem):
            pltpu.make_async_copy(
                pos_hbm.at[pl.ds(s_row + c * CH, CH)], buf, sem).start()

        for j in range(HALF):
            start_x(jnp.int32(j), bufs[j], sxs[j])
        for q in range(2):
            start_pos(jnp.int32(q), pbufs[q], sps[q])

        def body(i, carry):
            for j in range(UNROLL):
                g = UNROLL * i + j
                xb, sx, so = bufs[j % NR], sxs[j % NR], sos[j % NR]
                q = j // B
                pb, sp = pbufs[q], sps[q]
                c = 2 * i + q

                if j % B == 0:
                    pltpu.make_async_copy(
                        pos_hbm.at[pl.ds(s_row + c * CH, CH)], pb, sp).wait()

                pltpu.make_async_copy(
                    x_hbm.at[pl.ds(row_off(g), CH)], xb, sx).wait()

                U = 4

                def add_body(k, carry2):
                    sl = pl.ds(k * _L, _L)
                    for r0 in range(0, CH, U):
                        ps = [pb[r0 + t, sl] for t in range(U)]
                        for t in range(U):
                            plsc.addupdate(xb.at[r0 + t, sl], ps[t])
                    return carry2

                lax.fori_loop(0, D // _L, add_body, 0)

                pltpu.make_async_copy(
                    xb, out_hbm.at[pl.ds(row_off(g), CH)], so).start()

                j2 = (j + HALF) % NR
                g_old, g_new = g - HALF, g + HALF

                @pl.when(g_old >= 0)
                def _():
                    pltpu.make_async_copy(
                        bufs[j2], out_hbm.at[pl.ds(row_off(g_old), CH)],
                        sos[j2]).wait()

                @pl.when(g_new < G)
                def _():
                    start_x(g_new, bufs[j2], sxs[j2])

                if j % B == B - 1:
                    @pl.when(c + 2 < NCH)
                    def _():
                        start_pos(c + 2, pb, sp)
            return carry

        lax.fori_loop(0, G // UNROLL, body, 0)

        for j in range(HALF):
            g_last = G - HALF + j
            pltpu.make_async_copy(
                bufs[g_last % NR],
                out_hbm.at[pl.ds(row_off(jnp.int32(g_last)), CH)],
                sos[g_last % NR]).wait()

    return sc_add


def kernel(x, pos_table):
    B, S, D = x.shape
    out = _make_sc_add(B, S, D)(x.reshape(B * S, D), pos_table)
    return out.reshape(B, S, D)


# speedup vs baseline: 1.1057x; 1.1057x over previous
import functools

import jax
import jax.numpy as jnp
from jax import lax
from jax.experimental import pallas as pl
from jax.experimental.pallas import tpu as pltpu
from jax.experimental.pallas import tpu_sc as plsc

_NC, _NS, _L = 2, 16, 16
_NW = _NC * _NS


def _make_sc_add(B, S, D):
    CH = 8
    SPW = S // _NW
    NCH = SPW // CH
    G = NCH * B
    NR = 8
    UNROLL = 2 * B
    HALF = NR // 2
    assert S % _NW == 0 and SPW % CH == 0 and G % UNROLL == 0
    assert B == 4 and UNROLL % NR == 0 and D % _L == 0

    mesh = plsc.VectorSubcoreMesh(core_axis_name="c", subcore_axis_name="s")

    @functools.partial(
        pl.kernel,
        out_type=jax.ShapeDtypeStruct((B * S, D), jnp.float32),
        mesh=mesh,
        compiler_params=pltpu.CompilerParams(use_tc_tiling_on_sc=True),
        scratch_types=(
            [pltpu.VMEM((CH, D), jnp.float32) for _ in range(NR + 2)]
            + [pltpu.SemaphoreType.DMA for _ in range(2 * NR + 2)]
        ),
    )
    def sc_add(x_hbm, pos_hbm, out_hbm, *refs):
        bufs = refs[0:NR]
        pbufs = refs[NR:NR + 2]
        sxs = refs[NR + 2:2 * NR + 2]
        sos = refs[2 * NR + 2:3 * NR + 2]
        sps = refs[3 * NR + 2:3 * NR + 4]

        w = lax.axis_index("s") * _NC + lax.axis_index("c")
        s_row = w * SPW

        def row_off(g):
            return lax.rem(g, B) * S + s_row + lax.div(g, B) * CH

        def start_x(g, buf, sem):
            pltpu.make_async_copy(
                x_hbm.at[pl.ds(row_off(g), CH)], buf, sem).start()

        def start_pos(c, buf, sem):
            pltpu.make_async_copy(
                pos_hbm.at[pl.ds(s_row + c * CH, CH)], buf, sem).start()

        for j in range(HALF):
            start_x(jnp.int32(j), bufs[j], sxs[j])
        for q in range(2):
            start_pos(jnp.int32(q), pbufs[q], sps[q])

        def body(i, carry):
            for j in range(UNROLL):
                g = UNROLL * i + j
                xb, sx, so = bufs[j % NR], sxs[j % NR], sos[j % NR]
                q = j // B
                pb, sp = pbufs[q], sps[q]
                c = 2 * i + q

                if j % B == 0:
                    pltpu.make_async_copy(
                        pos_hbm.at[pl.ds(s_row + c * CH, CH)], pb, sp).wait()

                pltpu.make_async_copy(
                    x_hbm.at[pl.ds(row_off(g), CH)], xb, sx).wait()

                U = 8

                def add_body(k, carry2):
                    sl = pl.ds(k * _L, _L)
                    for r0 in range(0, CH, U):
                        ps = [pb[r0 + t, sl] for t in range(U)]
                        for t in range(U):
                            plsc.addupdate(xb.at[r0 + t, sl], ps[t])
                    return carry2

                lax.fori_loop(0, D // _L, add_body, 0)

                pltpu.make_async_copy(
                    xb, out_hbm.at[pl.ds(row_off(g), CH)], so).start()

                j2 = (j + HALF) % NR
                g_old, g_new = g - HALF, g + HALF

                @pl.when(g_old >= 0)
                def _():
                    pltpu.make_async_copy(
                        bufs[j2], out_hbm.at[pl.ds(row_off(g_old), CH)],
                        sos[j2]).wait()

                @pl.when(g_new < G)
                def _():
                    start_x(g_new, bufs[j2], sxs[j2])

                if j % B == B - 1:
                    @pl.when(c + 2 < NCH)
                    def _():
                        start_pos(c + 2, pb, sp)
            return carry

        lax.fori_loop(0, G // UNROLL, body, 0)

        for j in range(HALF):
            g_last = G - HALF + j
            pltpu.make_async_copy(
                bufs[g_last % NR],
                out_hbm.at[pl.ds(row_off(jnp.int32(g_last)), CH)],
                sos[g_last % NR]).wait()

    return sc_add


def kernel(x, pos_table):
    B, S, D = x.shape
    out = _make_sc_add(B, S, D)(x.reshape(B * S, D), pos_table)
    return out.reshape(B, S, D)
